# baseline (speedup 1.0000x reference)
import functools

import jax
import jax.numpy as jnp
from jax import lax
from jax.experimental import pallas as pl
from jax.experimental.pallas import tpu as pltpu
from jax.experimental.pallas import tpu_sc as plsc

NUM_FIELDS = 26
VOCAB = 100000
EMBED_DIM = 16
BATCH = 4096
SEQ = 20
POSITIONS = BATCH * SEQ

_info = plsc.get_sparse_core_info()
NC = _info.num_cores
NS = _info.num_subcores
NW = NC * NS
B_PER_W = POSITIONS // NW


def _make_sc_body(nf):
    def _sc_body(idx_hbm, tab_hbm, out_hbm, idx_v, rows_v, sem_i, sem_g, sem_o):
        wid = lax.axis_index("s") * NC + lax.axis_index("c")
        base = wid * B_PER_W

        def start_idx(f):
            b = f % 2
            return pltpu.async_copy(
                idx_hbm.at[f, pl.ds(base, B_PER_W)], idx_v.at[b], sem_i.at[b]
            )

        idx_cp = start_idx(0)
        outs = [None, None]
        for f in range(nf):
            b = f % 2
            next_idx = start_idx(f + 1) if f + 1 < nf else None
            idx_cp.wait()
            if outs[b] is not None:
                outs[b].wait()
            g = pltpu.async_copy(
                tab_hbm.at[f].at[idx_v.at[b]], rows_v.at[b], sem_g.at[b]
            )
            g.wait()
            outs[b] = pltpu.async_copy(
                rows_v.at[b],
                out_hbm.at[pl.ds(base, B_PER_W), pl.ds(f * EMBED_DIM, EMBED_DIM)],
                sem_o.at[b],
            )
            idx_cp = next_idx
        for o in outs:
            if o is not None:
                o.wait()

    return _sc_body


def _make_gather_kernel(nf):
    return pl.kernel(
        _make_sc_body(nf),
        out_type=jax.ShapeDtypeStruct((POSITIONS, nf * EMBED_DIM), jnp.float32),
        mesh=plsc.VectorSubcoreMesh(core_axis_name="c", subcore_axis_name="s"),
        scratch_types=[
            pltpu.VMEM((2, B_PER_W), jnp.int32),
            pltpu.VMEM((2, B_PER_W, EMBED_DIM), jnp.float32),
            pltpu.SemaphoreType.DMA((2,)),
            pltpu.SemaphoreType.DMA((2,)),
            pltpu.SemaphoreType.DMA((2,)),
        ],
        compiler_params=pltpu.CompilerParams(use_tc_tiling_on_sc=False),
    )


_GROUPS = (7, 7, 6, 6)
_group_kernels = [_make_gather_kernel(nf) for nf in _GROUPS]


@jax.jit
def kernel(indices, tables):
    idx2 = indices.reshape(NUM_FIELDS, POSITIONS)
    outs = []
    f0 = 0
    for nf, gk in zip(_GROUPS, _group_kernels):
        outs.append(gk(idx2[f0 : f0 + nf], tables[f0 : f0 + nf]))
        f0 += nf
    out = jnp.concatenate(outs, axis=-1)
    return out.reshape(BATCH, SEQ, NUM_FIELDS * EMBED_DIM)


# speedup vs baseline: 1.4092x; 1.4092x over previous
import jax
import jax.numpy as jnp
from jax import lax
from jax.experimental import pallas as pl
from jax.experimental.pallas import tpu as pltpu
from jax.experimental.pallas import tpu_sc as plsc

NF, V, D, B, S = 26, 100000, 16, 4096, 20
POS = B * S
VTAIL = V - (V // 128) * 128
VMAIN = V - VTAIL
VP = 100032
CW = 1024
CPF = VMAIN // CW + 1
NCH = NF * CPF
NW_ = 32
NIT = (NCH + NW_ - 1) // NW_

_info = plsc.get_sparse_core_info()
NC, NS = _info.num_cores, _info.num_subcores
NW = NC * NS
B_PER_W = POS // NW


def _repack_body(tab_hbm, tail_hbm, lin_hbm, slab_v, out_v, tail_v, tail2_v):
    core = lax.axis_index("c")
    tile = lax.axis_index("s")
    wid = tile * NC + core
    lanes = lax.iota(jnp.int32, 16)
    g_rb = lanes // 8
    g_r = lanes % 8

    def chunk_body(i, _):
        ch = jnp.minimum(i * NW_ + wid, NCH - 1)
        f = ch // CPF
        k = ch % CPF
        c0 = pl.multiple_of(jnp.minimum(k * CW, VMAIN - CW), 128)
        pltpu.sync_copy(tab_hbm.at[f, pl.ds(0, 8), pl.ds(c0, CW)],
                        slab_v.at[0])
        pltpu.sync_copy(tab_hbm.at[f, pl.ds(8, 8), pl.ds(c0, CW)],
                        slab_v.at[1])

        def repack(j, _):
            col = plsc.load_gather(
                slab_v, [g_rb, g_r, jnp.full((16,), j, jnp.int32)])
            out_v[j // 8, pl.ds((j % 8) * D, D)] = col
            return 0

        lax.fori_loop(0, CW, repack, 0, unroll=4)
        pltpu.sync_copy(out_v, lin_hbm.at[pl.ds(pl.multiple_of((f * VP + c0) // 8, 8), CW // 8), :])
        return 0

    lax.fori_loop(0, NIT, chunk_body, 0)

    @pl.when(wid < NF)
    def _():
        f = wid
        pltpu.sync_copy(tail_hbm.at[f], tail_v)

        def tail_repack(j, _):
            tail2_v[j // 8, pl.ds((j % 8) * D, D)] = tail_v[j, pl.ds(0, D)]
            return 0

        lax.fori_loop(0, VTAIL, tail_repack, 0)
        pltpu.sync_copy(
            tail2_v, lin_hbm.at[pl.ds(pl.multiple_of((f * VP + VMAIN) // 8, 8), 8), :])


_repack = pl.kernel(
    _repack_body,
    out_type=jax.ShapeDtypeStruct((NF * VP // 8, 128), jnp.float32),
    mesh=plsc.VectorSubcoreMesh(core_axis_name="c", subcore_axis_name="s"),
    scratch_types=[
        pltpu.VMEM((2, 8, CW), jnp.float32),
        pltpu.VMEM((CW // 8, 128), jnp.float32),
        pltpu.VMEM((VTAIL, D), jnp.float32),
        pltpu.VMEM((8, 128), jnp.float32),
    ],
    compiler_params=pltpu.CompilerParams(needs_layout_passes=False),
)


def _gather_body(idx_hbm, tab_hbm, out_hbm, idx_v, idxg_v, rows_v, sem_i, sem_g, sem_o):
    wid = lax.axis_index("s") * NC + lax.axis_index("c")
    base = wid * B_PER_W

    def start_idx(f):
        b = f % 2
        return pltpu.async_copy(
            idx_hbm.at[f, pl.ds(base, B_PER_W)], idx_v.at[b], sem_i.at[b]
        )

    idx_cp = start_idx(0)
    outs = [None, None]
    for f in range(NF):
        b = f % 2
        next_idx = start_idx(f + 1) if f + 1 < NF else None
        idx_cp.wait()

        def add_off(i, _):
            idxg_v[b, pl.ds(i * 16, 16)] = idx_v[b, pl.ds(i * 16, 16)] + f * VP
            return 0

        lax.fori_loop(0, B_PER_W // 16, add_off, 0, unroll=4)
        if outs[b] is not None:
            outs[b].wait()
        g = pltpu.async_copy(tab_hbm.at[idxg_v.at[b]], rows_v.at[b],
                             sem_g.at[b])
        g.wait()
        outs[b] = pltpu.async_copy(
            rows_v.at[b],
            out_hbm.at[pl.ds(base, B_PER_W), pl.ds(f * D, D)],
            sem_o.at[b],
        )
        idx_cp = next_idx
    outs[0].wait()
    outs[1].wait()


_gather = pl.kernel(
    _gather_body,
    out_type=jax.ShapeDtypeStruct((POS, NF * D), jnp.float32),
    mesh=plsc.VectorSubcoreMesh(core_axis_name="c", subcore_axis_name="s"),
    scratch_types=[
        pltpu.VMEM((2, B_PER_W), jnp.int32),
        pltpu.VMEM((2, B_PER_W), jnp.int32),
        pltpu.VMEM((2, B_PER_W, D), jnp.float32),
        pltpu.SemaphoreType.DMA((2,)),
        pltpu.SemaphoreType.DMA((2,)),
        pltpu.SemaphoreType.DMA((2,)),
    ],
    compiler_params=pltpu.CompilerParams(use_tc_tiling_on_sc=False),
)


@jax.jit
def kernel(indices, tables):
    idx2 = indices.reshape(NF, POS)
    tab_t = jnp.transpose(tables, (0, 2, 1))
    tab_tail = tables[:, VMAIN:, :]
    lin = _repack(tab_t, tab_tail)
    tab_lin = lin.reshape(NF * VP, D)
    out = _gather(idx2, tab_lin)
    return out.reshape(B, S, NF * D)


# speedup vs baseline: 1.6157x; 1.1466x over previous
import jax
import jax.numpy as jnp
from jax import lax
from jax.experimental import pallas as pl
from jax.experimental.pallas import tpu as pltpu
from jax.experimental.pallas import tpu_sc as plsc

NF, V, D, B, S = 26, 100000, 16, 4096, 20
POS = B * S
VTAIL = V - (V // 128) * 128
VMAIN = V - VTAIL
VP = 100032
CW = 1024
CPF = VMAIN // CW + 1
NCH = NF * CPF
NW_ = 32
NIT = (NCH + NW_ - 1) // NW_

_info = plsc.get_sparse_core_info()
NC, NS = _info.num_cores, _info.num_subcores
NW = NC * NS
B_PER_W = POS // NW


def _repack_body(tab_hbm, tail_hbm, lin_hbm, slab_v, out_v, tail_v, tail2_v,
                 sem_s, sem_o):
    core = lax.axis_index("c")
    tile = lax.axis_index("s")
    wid = tile * NC + core
    lanes = lax.iota(jnp.int32, 16)
    g_rb = lanes // 8
    g_r = lanes % 8

    def addr(ch):
        ch = jnp.minimum(ch, NCH - 1)
        f = ch // CPF
        k = ch % CPF
        c0 = pl.multiple_of(jnp.minimum(k * CW, VMAIN - CW), 128)
        return f, c0

    def slab_copies(i, p):
        f, c0 = addr(i * NW_ + wid)
        return (
            pltpu.make_async_copy(
                tab_hbm.at[f, pl.ds(0, 8), pl.ds(c0, CW)], slab_v.at[p, 0],
                sem_s.at[p, 0]),
            pltpu.make_async_copy(
                tab_hbm.at[f, pl.ds(8, 8), pl.ds(c0, CW)], slab_v.at[p, 1],
                sem_s.at[p, 1]),
        )

    def out_copy(i, p):
        f, c0 = addr(i * NW_ + wid)
        row0 = pl.multiple_of((f * VP + c0) // 8, 8)
        return pltpu.make_async_copy(
            out_v.at[p], lin_hbm.at[pl.ds(row0, CW // 8), :], sem_o.at[p])

    for cp in slab_copies(0, 0):
        cp.start()
    for cp in slab_copies(1, 1):
        cp.start()

    def chunk_body(i, _):
        p = i % 2
        for cp in slab_copies(i, p):
            cp.wait()

        @pl.when(i >= 2)
        def _():
            out_copy(i - 2, p).wait()

        def repack8(j8, _):
            for r in range(8):
                col = plsc.load_gather(
                    slab_v.at[p],
                    [g_rb, g_r, jnp.full((16,), j8 * 8 + r, jnp.int32)])
                out_v[p, j8, pl.ds(r * D, D)] = col
            return 0

        lax.fori_loop(0, CW // 8, repack8, 0, unroll=2)
        out_copy(i, p).start()

        @pl.when(i + 2 < NIT)
        def _():
            for cp in slab_copies(i + 2, p):
                cp.start()

        return 0

    lax.fori_loop(0, NIT, chunk_body, 0)
    out_copy(NIT - 2, (NIT - 2) % 2).wait()
    out_copy(NIT - 1, (NIT - 1) % 2).wait()

    @pl.when(wid < NF)
    def _():
        f = wid
        pltpu.sync_copy(tail_hbm.at[f], tail_v)

        def tail_repack(j, _):
            tail2_v[j // 8, pl.ds((j % 8) * D, D)] = tail_v[j, pl.ds(0, D)]
            return 0

        lax.fori_loop(0, VTAIL, tail_repack, 0)
        pltpu.sync_copy(
            tail2_v, lin_hbm.at[pl.ds(pl.multiple_of((f * VP + VMAIN) // 8, 8), 8), :])


_repack = pl.kernel(
    _repack_body,
    out_type=jax.ShapeDtypeStruct((NF * VP // 8, 128), jnp.float32),
    mesh=plsc.VectorSubcoreMesh(core_axis_name="c", subcore_axis_name="s"),
    scratch_types=[
        pltpu.VMEM((2, 2, 8, CW), jnp.float32),
        pltpu.VMEM((2, CW // 8, 128), jnp.float32),
        pltpu.VMEM((VTAIL, D), jnp.float32),
        pltpu.VMEM((8, 128), jnp.float32),
        pltpu.SemaphoreType.DMA((2, 2)),
        pltpu.SemaphoreType.DMA((2,)),
    ],
    compiler_params=pltpu.CompilerParams(needs_layout_passes=False),
)


def _gather_body(idx_hbm, tab_hbm, out_hbm, idx_v, idxg_v, rows_v, sem_i, sem_g, sem_o):
    wid = lax.axis_index("s") * NC + lax.axis_index("c")
    base = wid * B_PER_W

    def start_idx(f):
        b = f % 2
        return pltpu.async_copy(
            idx_hbm.at[f, pl.ds(base, B_PER_W)], idx_v.at[b], sem_i.at[b]
        )

    idx_cp = start_idx(0)
    outs = [None, None]
    for f in range(NF):
        b = f % 2
        next_idx = start_idx(f + 1) if f + 1 < NF else None
        idx_cp.wait()

        def add_off(i, _):
            idxg_v[b, pl.ds(i * 16, 16)] = idx_v[b, pl.ds(i * 16, 16)] + f * VP
            return 0

        lax.fori_loop(0, B_PER_W // 16, add_off, 0, unroll=4)
        if outs[b] is not None:
            outs[b].wait()
        g = pltpu.async_copy(tab_hbm.at[idxg_v.at[b]], rows_v.at[b],
                             sem_g.at[b])
        g.wait()
        outs[b] = pltpu.async_copy(
            rows_v.at[b],
            out_hbm.at[pl.ds(base, B_PER_W), pl.ds(f * D, D)],
            sem_o.at[b],
        )
        idx_cp = next_idx
    outs[0].wait()
    outs[1].wait()


_gather = pl.kernel(
    _gather_body,
    out_type=jax.ShapeDtypeStruct((POS, NF * D), jnp.float32),
    mesh=plsc.VectorSubcoreMesh(core_axis_name="c", subcore_axis_name="s"),
    scratch_types=[
        pltpu.VMEM((2, B_PER_W), jnp.int32),
        pltpu.VMEM((2, B_PER_W), jnp.int32),
        pltpu.VMEM((2, B_PER_W, D), jnp.float32),
        pltpu.SemaphoreType.DMA((2,)),
        pltpu.SemaphoreType.DMA((2,)),
        pltpu.SemaphoreType.DMA((2,)),
    ],
    compiler_params=pltpu.CompilerParams(use_tc_tiling_on_sc=False),
)


@jax.jit
def kernel(indices, tables):
    idx2 = indices.reshape(NF, POS)
    tab_t = jnp.transpose(tables, (0, 2, 1))
    tab_tail = tables[:, VMAIN:, :]
    lin = _repack(tab_t, tab_tail)
    tab_lin = lin.reshape(NF * VP, D)
    out = _gather(idx2, tab_lin)
    return out.reshape(B, S, NF * D)
